# baseline (speedup 1.0000x reference)
import functools

import jax
import jax.numpy as jnp
from jax import lax
from jax.experimental import pallas as pl
from jax.experimental.pallas import tpu as pltpu
from jax.experimental.pallas import tpu_sc as plsc

N = 10000
D_IN = 128
D_H = 16
N_CLS = 10
E = 320000

NC = 2
NS = 16
NW = NC * NS
CW = 128
E_TOT = E + N
CHUNKS = -(-E_TOT // (NW * CW))
E_PAD = NW * CHUNKS * CW
PH = -(-CHUNKS // 2)
N_ACC = N + NS
ROWS_PER_SUB = N_ACC // NS

_f32 = jnp.float32


def _tc1_body(x_ref, w_ref, a2_ref, h_ref, st_ref):
    h = jnp.dot(x_ref[...], w_ref[...], preferred_element_type=_f32)
    h_ref[0:N, :] = h
    h_ref[N:N_ACC, :] = jnp.zeros((NS, D_H), _f32)
    st_ref[0:N, :] = jnp.dot(h, a2_ref[...], preferred_element_type=_f32)
    st_ref[N:N_ACC, :] = jnp.zeros((NS, 2), _f32)


def _tc_dense1(x_pad, W1, a2):
    return pl.pallas_call(
        _tc1_body,
        out_shape=[
            jax.ShapeDtypeStruct((N_ACC, D_H), _f32),
            jax.ShapeDtypeStruct((N_ACC, 2), _f32),
        ],
    )(x_pad, W1, a2)


def _tc2_body(acc_ref, b_ref, w_ref, a2_ref, h_ref, st_ref):
    acc = acc_ref[0] + acc_ref[1]
    num = acc[:, 0:D_H]
    den = acc[:, D_H:D_H + 1]
    o = num / (den + 1e-16) + b_ref[...]
    o = jnp.maximum(o, 0.0)
    h = jnp.dot(o, w_ref[...], preferred_element_type=_f32)
    h_ref[...] = h
    st_ref[...] = jnp.dot(h, a2_ref[...], preferred_element_type=_f32)


def _tc_dense2(acc, b1, W2, a2):
    return pl.pallas_call(
        _tc2_body,
        out_shape=[
            jax.ShapeDtypeStruct((N_ACC, D_H), _f32),
            jax.ShapeDtypeStruct((N_ACC, 2), _f32),
        ],
    )(acc, b1, W2, a2)


def _tc3_body(acc_ref, b_ref, wc_ref, bc_ref, out_ref):
    acc = acc_ref[0] + acc_ref[1]
    o = acc[:, 0:D_H] / (acc[:, D_H:D_H + 1] + 1e-16) + b_ref[...]
    logits = jnp.dot(o, wc_ref[...], preferred_element_type=_f32) + bc_ref[...]
    m = jnp.max(logits, axis=1, keepdims=True)
    z = logits - m
    lse = jnp.log(jnp.sum(jnp.exp(z), axis=1, keepdims=True))
    out_ref[...] = z - lse


def _tc_dense3(acc, b2, Wc, bc):
    return pl.pallas_call(
        _tc3_body,
        out_shape=jax.ShapeDtypeStruct((N_ACC, N_CLS), _f32),
    )(acc, b2, Wc, bc)


def _sc_body(src_hbm, dst_hbm, st_hbm, h_hbm, zeros_hbm, acc_out,
             src_v, dst_v, st_v, rows_v, rows32_v, w_c, acc_sh,
             gsem0, gsem1, ssem0, ssem1):
    cid = lax.axis_index("c")
    sid = lax.axis_index("s")
    wid = sid * NC + cid

    pltpu.sync_copy(st_hbm, st_v)
    pltpu.sync_copy(zeros_hbm.at[pl.ds(sid * ROWS_PER_SUB, ROWS_PER_SUB)],
                    acc_sh.at[pl.ds(sid * ROWS_PER_SUB, ROWS_PER_SUB)])

    zeros16 = jnp.zeros((16,), jnp.int32)
    ones16 = jnp.ones((16,), jnp.int32)
    gsems = (gsem0, gsem1)
    ssems = (ssem0, ssem1)

    def _process(jj, b, count):
        pltpu.make_async_copy(h_hbm.at[src_v.at[jj]], rows_v.at[b],
                              gsems[b]).wait()
        nb = 1 - b

        @pl.when(jj < count - 1)
        def _():
            pltpu.async_copy(h_hbm.at[src_v.at[jj + 1]], rows_v.at[nb],
                             gsems[nb])

        @pl.when(jj >= 2)
        def _():
            pltpu.make_async_copy(rows32_v.at[b],
                                  acc_sh.at[dst_v.at[jj - 2]],
                                  ssems[b]).wait()

        for k in range(CW // 16):
            srcs = src_v[jj, pl.ds(k * 16, 16)]
            dsts = dst_v[jj, pl.ds(k * 16, 16)]
            sv = plsc.load_gather(st_v, [srcs, zeros16])
            tv = plsc.load_gather(st_v, [dsts, ones16])
            z = sv + tv
            z = jnp.where(z >= 0.0, z, 0.2 * z)
            w_c[pl.ds(k * 16, 16)] = jnp.exp(z)
        for r in range(CW):
            wb = plsc.load_gather(w_c, [jnp.full((16,), r, jnp.int32)])
            rows32_v[b, r, 0:D_H] = rows_v[b, r, :] * wb
            rows32_v[b, r, D_H:2 * D_H] = wb
        pltpu.async_copy(rows32_v.at[b], acc_sh.at[dst_v.at[jj]],
                         ssems[b], add=True)

    def _run_phase(start, count):
        pltpu.sync_copy(src_hbm.at[wid].at[pl.ds(start, count)],
                        src_v.at[pl.ds(0, count)])
        pltpu.sync_copy(dst_hbm.at[wid].at[pl.ds(start, count)],
                        dst_v.at[pl.ds(0, count)])
        pltpu.async_copy(h_hbm.at[src_v.at[0]], rows_v.at[0], gsem0)

        def cbody(p, carry):
            _process(2 * p, 0, count)
            _process(2 * p + 1, 1, count)
            return carry

        lax.fori_loop(0, count // 2, cbody, 0)
        if count % 2:
            _process(count - 1, 0, count)
            last0, last1 = count - 1, count - 2
        else:
            last0, last1 = count - 2, count - 1
        pltpu.make_async_copy(rows32_v.at[0], acc_sh.at[dst_v.at[last0]],
                              ssem0).wait()
        pltpu.make_async_copy(rows32_v.at[1], acc_sh.at[dst_v.at[last1]],
                              ssem1).wait()

    _run_phase(0, PH)
    _run_phase(PH, CHUNKS - PH)
    plsc.subcore_barrier()

    pltpu.sync_copy(acc_sh.at[pl.ds(sid * ROWS_PER_SUB, ROWS_PER_SUB)],
                    acc_out.at[cid].at[pl.ds(sid * ROWS_PER_SUB, ROWS_PER_SUB)])


_sc_layer = functools.partial(
    pl.kernel,
    out_type=jax.ShapeDtypeStruct((NC, N_ACC, 2 * D_H), _f32),
    mesh=plsc.VectorSubcoreMesh(core_axis_name="c", subcore_axis_name="s"),
    compiler_params=pltpu.CompilerParams(needs_layout_passes=False,
                                         use_tc_tiling_on_sc=False),
    scratch_types=[
        pltpu.VMEM((PH, CW), jnp.int32),
        pltpu.VMEM((PH, CW), jnp.int32),
        pltpu.VMEM((N_ACC, 2), _f32),
        pltpu.VMEM((2, CW, D_H), _f32),
        pltpu.VMEM((2, CW, 2 * D_H), _f32),
        pltpu.VMEM((CW,), _f32),
        pltpu.VMEM_SHARED((N_ACC, 2 * D_H), _f32),
        pltpu.SemaphoreType.DMA,
        pltpu.SemaphoreType.DMA,
        pltpu.SemaphoreType.DMA,
        pltpu.SemaphoreType.DMA,
    ],
)(_sc_body)


def kernel(x, edge_index, edge_attr, W1, a1_src, a1_dst, b1, W2, a2_src,
           a2_dst, b2, Wc, bc):
    del edge_attr
    src = edge_index[0].astype(jnp.int32)
    dst = edge_index[1].astype(jnp.int32)
    loop_idx = jnp.arange(N, dtype=jnp.int32)
    pad_idx = N + (jnp.arange(E_PAD - E_TOT, dtype=jnp.int32) % NS)
    src_all = jnp.concatenate([src, loop_idx, pad_idx]).reshape(NW, CHUNKS, CW)
    dst_all = jnp.concatenate([dst, loop_idx, pad_idx]).reshape(NW, CHUNKS, CW)

    zeros = jnp.zeros((N_ACC, 2 * D_H), _f32)

    a1 = jnp.concatenate([a1_src[:, None], a1_dst[:, None]], axis=1)
    a2 = jnp.concatenate([a2_src[:, None], a2_dst[:, None]], axis=1)

    h1, st1 = _tc_dense1(x, W1, a1)
    acc1 = _sc_layer(src_all, dst_all, st1, h1, zeros)
    h2, st2 = _tc_dense2(acc1, b1[None, :], W2, a2)
    acc2 = _sc_layer(src_all, dst_all, st2, h2, zeros)
    out = _tc_dense3(acc2, b2[None, :], Wc, bc[None, :])
    return out[:N]


# speedup vs baseline: 1.0797x; 1.0797x over previous
import functools

import jax
import jax.numpy as jnp
from jax import lax
from jax.experimental import pallas as pl
from jax.experimental.pallas import tpu as pltpu
from jax.experimental.pallas import tpu_sc as plsc

N = 10000
D_IN = 128
D_H = 16
N_CLS = 10
E = 320000

NC = 2
NS = 16
NW = NC * NS
CW = 128
E_TOT = E + N
CHUNKS = -(-E_TOT // (NW * CW))
E_PAD = NW * CHUNKS * CW
PH = -(-CHUNKS // 2)
N_ACC = N + NS
ROWS_PER_SUB = N_ACC // NS

_f32 = jnp.float32


def _tc1_body(x_ref, w_ref, a2_ref, h_ref, st_ref):
    h = jnp.dot(x_ref[...], w_ref[...], preferred_element_type=_f32)
    h_ref[0:N, :] = h
    h_ref[N:N_ACC, :] = jnp.zeros((NS, D_H), _f32)
    st_ref[0:N, :] = jnp.dot(h, a2_ref[...], preferred_element_type=_f32)
    st_ref[N:N_ACC, :] = jnp.zeros((NS, 2), _f32)


def _tc_dense1(x_pad, W1, a2):
    return pl.pallas_call(
        _tc1_body,
        out_shape=[
            jax.ShapeDtypeStruct((N_ACC, D_H), _f32),
            jax.ShapeDtypeStruct((N_ACC, 2), _f32),
        ],
    )(x_pad, W1, a2)


def _tc2_body(acc_ref, b_ref, w_ref, a2_ref, h_ref, st_ref):
    acc = acc_ref[0] + acc_ref[1]
    num = acc[:, 0:D_H]
    den = acc[:, D_H:D_H + 1]
    o = num / (den + 1e-16) + b_ref[...]
    o = jnp.maximum(o, 0.0)
    h = jnp.dot(o, w_ref[...], preferred_element_type=_f32)
    h_ref[...] = h
    st_ref[...] = jnp.dot(h, a2_ref[...], preferred_element_type=_f32)


def _tc_dense2(acc, b1, W2, a2):
    return pl.pallas_call(
        _tc2_body,
        out_shape=[
            jax.ShapeDtypeStruct((N_ACC, D_H), _f32),
            jax.ShapeDtypeStruct((N_ACC, 2), _f32),
        ],
    )(acc, b1, W2, a2)


def _tc3_body(acc_ref, b_ref, wc_ref, bc_ref, out_ref):
    acc = acc_ref[0] + acc_ref[1]
    o = acc[:, 0:D_H] / (acc[:, D_H:D_H + 1] + 1e-16) + b_ref[...]
    logits = jnp.dot(o, wc_ref[...], preferred_element_type=_f32) + bc_ref[...]
    m = jnp.max(logits, axis=1, keepdims=True)
    z = logits - m
    lse = jnp.log(jnp.sum(jnp.exp(z), axis=1, keepdims=True))
    out_ref[...] = z - lse


def _tc_dense3(acc, b2, Wc, bc):
    return pl.pallas_call(
        _tc3_body,
        out_shape=jax.ShapeDtypeStruct((N_ACC, N_CLS), _f32),
    )(acc, b2, Wc, bc)


def _sc_body(src_hbm, dst_hbm, st_hbm, h_hbm, zeros_hbm, acc_out,
             src_v, dst_v, st_v, rows_v, rows32_v, acc_sh,
             gsem0, gsem1, ssem0, ssem1):
    cid = lax.axis_index("c")
    sid = lax.axis_index("s")
    wid = sid * NC + cid

    pltpu.sync_copy(st_hbm, st_v)
    pltpu.sync_copy(zeros_hbm.at[pl.ds(sid * ROWS_PER_SUB, ROWS_PER_SUB)],
                    acc_sh.at[pl.ds(sid * ROWS_PER_SUB, ROWS_PER_SUB)])

    zeros16 = jnp.zeros((16,), jnp.int32)
    ones16 = jnp.ones((16,), jnp.int32)
    gsems = (gsem0, gsem1)
    ssems = (ssem0, ssem1)

    def _process(jj, b, count):
        pltpu.make_async_copy(h_hbm.at[src_v.at[jj]], rows_v.at[b],
                              gsems[b]).wait()
        nb = 1 - b

        @pl.when(jj < count - 1)
        def _():
            pltpu.async_copy(h_hbm.at[src_v.at[jj + 1]], rows_v.at[nb],
                             gsems[nb])

        @pl.when(jj >= 2)
        def _():
            pltpu.make_async_copy(rows32_v.at[b],
                                  acc_sh.at[dst_v.at[jj - 2]],
                                  ssems[b]).wait()

        for k in range(CW // 16):
            srcs = src_v[jj, pl.ds(k * 16, 16)]
            dsts = dst_v[jj, pl.ds(k * 16, 16)]
            sv = plsc.load_gather(st_v, [srcs, zeros16])
            tv = plsc.load_gather(st_v, [dsts, ones16])
            z = sv + tv
            z = jnp.where(z >= 0.0, z, 0.2 * z)
            wv = jnp.exp(z)
            plsc.store_scatter(
                rows32_v.at[b],
                [jnp.arange(k * 16, k * 16 + 16, dtype=jnp.int32),
                 jnp.full((16,), D_H, jnp.int32)], wv)
            for i in range(16):
                wsc = wv[i]
                r = k * 16 + i
                rows32_v[b, r, 0:D_H] = rows_v[b, r, :] * wsc
        pltpu.async_copy(rows32_v.at[b], acc_sh.at[dst_v.at[jj]],
                         ssems[b], add=True)

    def _run_phase(start, count):
        pltpu.sync_copy(src_hbm.at[wid].at[pl.ds(start, count)],
                        src_v.at[pl.ds(0, count)])
        pltpu.sync_copy(dst_hbm.at[wid].at[pl.ds(start, count)],
                        dst_v.at[pl.ds(0, count)])
        pltpu.async_copy(h_hbm.at[src_v.at[0]], rows_v.at[0], gsem0)

        def cbody(p, carry):
            _process(2 * p, 0, count)
            _process(2 * p + 1, 1, count)
            return carry

        lax.fori_loop(0, count // 2, cbody, 0)
        if count % 2:
            _process(count - 1, 0, count)
            last0, last1 = count - 1, count - 2
        else:
            last0, last1 = count - 2, count - 1
        pltpu.make_async_copy(rows32_v.at[0], acc_sh.at[dst_v.at[last0]],
                              ssem0).wait()
        pltpu.make_async_copy(rows32_v.at[1], acc_sh.at[dst_v.at[last1]],
                              ssem1).wait()

    _run_phase(0, PH)
    _run_phase(PH, CHUNKS - PH)
    plsc.subcore_barrier()

    pltpu.sync_copy(acc_sh.at[pl.ds(sid * ROWS_PER_SUB, ROWS_PER_SUB)],
                    acc_out.at[cid].at[pl.ds(sid * ROWS_PER_SUB, ROWS_PER_SUB)])


_sc_layer = functools.partial(
    pl.kernel,
    out_type=jax.ShapeDtypeStruct((NC, N_ACC, 2 * D_H), _f32),
    mesh=plsc.VectorSubcoreMesh(core_axis_name="c", subcore_axis_name="s"),
    compiler_params=pltpu.CompilerParams(needs_layout_passes=False,
                                         use_tc_tiling_on_sc=False),
    scratch_types=[
        pltpu.VMEM((PH, CW), jnp.int32),
        pltpu.VMEM((PH, CW), jnp.int32),
        pltpu.VMEM((N_ACC, 2), _f32),
        pltpu.VMEM((2, CW, D_H), _f32),
        pltpu.VMEM((2, CW, 2 * D_H), _f32),
        pltpu.VMEM_SHARED((N_ACC, 2 * D_H), _f32),
        pltpu.SemaphoreType.DMA,
        pltpu.SemaphoreType.DMA,
        pltpu.SemaphoreType.DMA,
        pltpu.SemaphoreType.DMA,
    ],
)(_sc_body)


def kernel(x, edge_index, edge_attr, W1, a1_src, a1_dst, b1, W2, a2_src,
           a2_dst, b2, Wc, bc):
    del edge_attr
    src = edge_index[0].astype(jnp.int32)
    dst = edge_index[1].astype(jnp.int32)
    loop_idx = jnp.arange(N, dtype=jnp.int32)
    pad_idx = N + (jnp.arange(E_PAD - E_TOT, dtype=jnp.int32) % NS)
    src_all = jnp.concatenate([src, loop_idx, pad_idx]).reshape(NW, CHUNKS, CW)
    dst_all = jnp.concatenate([dst, loop_idx, pad_idx]).reshape(NW, CHUNKS, CW)

    zeros = jnp.zeros((N_ACC, 2 * D_H), _f32)

    a1 = jnp.concatenate([a1_src[:, None], a1_dst[:, None]], axis=1)
    a2 = jnp.concatenate([a2_src[:, None], a2_dst[:, None]], axis=1)

    h1, st1 = _tc_dense1(x, W1, a1)
    acc1 = _sc_layer(src_all, dst_all, st1, h1, zeros)
    h2, st2 = _tc_dense2(acc1, b1[None, :], W2, a2)
    acc2 = _sc_layer(src_all, dst_all, st2, h2, zeros)
    out = _tc_dense3(acc2, b2[None, :], Wc, bc[None, :])
    return out[:N]


# speedup vs baseline: 1.2363x; 1.1450x over previous
import functools

import jax
import jax.numpy as jnp
from jax import lax
from jax.experimental import pallas as pl
from jax.experimental.pallas import tpu as pltpu
from jax.experimental.pallas import tpu_sc as plsc

N = 10000
D_IN = 128
D_H = 16
N_CLS = 10
E = 320000

NC = 2
NS = 16
NW = NC * NS
CW = 128
E_TOT = E + N
CHUNKS = -(-E_TOT // (NW * CW))
E_PAD = NW * CHUNKS * CW
PH = -(-CHUNKS // 2)
N_ACC = 10112
N_DISC = N_ACC - N
ROWS_PER_SUB = N_ACC // NS

_f32 = jnp.float32


def _tc1_body(x_ref, w_ref, a2_ref, h_ref, st_ref):
    h = jnp.dot(x_ref[...], w_ref[...], preferred_element_type=_f32)
    h_ref[0:N, :] = h
    h_ref[N:N_ACC, :] = jnp.zeros((N_DISC, D_H), _f32)
    st_ref[0:N, :] = jnp.dot(h, a2_ref[...], preferred_element_type=_f32)
    st_ref[N:N_ACC, :] = jnp.zeros((N_DISC, 2), _f32)


def _tc_dense1(x_pad, W1, a2):
    return pl.pallas_call(
        _tc1_body,
        out_shape=[
            jax.ShapeDtypeStruct((N_ACC, D_H), _f32),
            jax.ShapeDtypeStruct((N_ACC, 2), _f32),
        ],
    )(x_pad, W1, a2)


def _tc2_body(accn_ref, accd_ref, b_ref, w_ref, a2_ref, h_ref, st_ref):
    num = accn_ref[0] + accn_ref[1]
    den = accd_ref[0] + accd_ref[1]
    o = num / (den + 1e-16) + b_ref[...]
    o = jnp.maximum(o, 0.0)
    h = jnp.dot(o, w_ref[...], preferred_element_type=_f32)
    h_ref[...] = h
    st_ref[...] = jnp.dot(h, a2_ref[...], preferred_element_type=_f32)


def _tc_dense2(accn, accd, b1, W2, a2):
    return pl.pallas_call(
        _tc2_body,
        out_shape=[
            jax.ShapeDtypeStruct((N_ACC, D_H), _f32),
            jax.ShapeDtypeStruct((N_ACC, 2), _f32),
        ],
    )(accn, accd, b1, W2, a2)


def _tc3_body(accn_ref, accd_ref, b_ref, wc_ref, bc_ref, out_ref):
    num = accn_ref[0] + accn_ref[1]
    den = accd_ref[0] + accd_ref[1]
    o = num / (den + 1e-16) + b_ref[...]
    logits = jnp.dot(o, wc_ref[...], preferred_element_type=_f32) + bc_ref[...]
    m = jnp.max(logits, axis=1, keepdims=True)
    z = logits - m
    lse = jnp.log(jnp.sum(jnp.exp(z), axis=1, keepdims=True))
    out_ref[...] = z - lse


def _tc_dense3(accn, accd, b2, Wc, bc):
    return pl.pallas_call(
        _tc3_body,
        out_shape=jax.ShapeDtypeStruct((N_ACC, N_CLS), _f32),
    )(accn, accd, b2, Wc, bc)


def _sc_body(src_hbm, dst_hbm, st_hbm, h_hbm, zn_hbm, zd_hbm,
             accn_out, accd_out,
             src_v, dst_v, st_v, rows_v, rows_s, w_v,
             h_sh, accn_sh, accd_sh,
             gsem0, gsem1, nsem0, nsem1, dsem0, dsem1):
    cid = lax.axis_index("c")
    sid = lax.axis_index("s")
    wid = sid * NC + cid
    sub_rows = pl.ds(sid * ROWS_PER_SUB, ROWS_PER_SUB)

    pltpu.sync_copy(st_hbm, st_v)
    pltpu.sync_copy(h_hbm.at[sub_rows], h_sh.at[sub_rows])
    pltpu.sync_copy(zn_hbm.at[sub_rows], accn_sh.at[sub_rows])
    pltpu.sync_copy(zd_hbm.at[sub_rows], accd_sh.at[sub_rows])
    plsc.subcore_barrier()

    zeros16 = jnp.zeros((16,), jnp.int32)
    ones16 = jnp.ones((16,), jnp.int32)
    gsems = (gsem0, gsem1)
    nsems = (nsem0, nsem1)
    dsems = (dsem0, dsem1)

    def _process(jj, b, count):
        pltpu.make_async_copy(h_sh.at[src_v.at[jj]], rows_v.at[b],
                              gsems[b]).wait()
        nb = 1 - b

        @pl.when(jj < count - 1)
        def _():
            pltpu.async_copy(h_sh.at[src_v.at[jj + 1]], rows_v.at[nb],
                             gsems[nb])

        @pl.when(jj >= 2)
        def _():
            pltpu.make_async_copy(rows_s.at[b], accn_sh.at[dst_v.at[jj - 2]],
                                  nsems[b]).wait()
            pltpu.make_async_copy(w_v.at[b], accd_sh.at[dst_v.at[jj - 2]],
                                  dsems[b]).wait()

        for k in range(CW // 16):
            srcs = src_v[jj, pl.ds(k * 16, 16)]
            dsts = dst_v[jj, pl.ds(k * 16, 16)]
            sv = plsc.load_gather(st_v, [srcs, zeros16])
            tv = plsc.load_gather(st_v, [dsts, ones16])
            z = sv + tv
            z = jnp.where(z >= 0.0, z, 0.2 * z)
            wv = jnp.exp(z)
            w_v[b, pl.ds(k * 16, 16)] = wv
            for i in range(16):
                wsc = wv[i]
                r = k * 16 + i
                rows_s[b, r, :] = rows_v[b, r, :] * wsc
        pltpu.async_copy(rows_s.at[b], accn_sh.at[dst_v.at[jj]],
                         nsems[b], add=True)
        pltpu.async_copy(w_v.at[b], accd_sh.at[dst_v.at[jj]],
                         dsems[b], add=True)

    def _run_phase(start, count):
        pltpu.sync_copy(src_hbm.at[wid].at[pl.ds(start, count)],
                        src_v.at[pl.ds(0, count)])
        pltpu.sync_copy(dst_hbm.at[wid].at[pl.ds(start, count)],
                        dst_v.at[pl.ds(0, count)])
        pltpu.async_copy(h_sh.at[src_v.at[0]], rows_v.at[0], gsem0)

        def cbody(p, carry):
            _process(2 * p, 0, count)
            _process(2 * p + 1, 1, count)
            return carry

        lax.fori_loop(0, count // 2, cbody, 0)
        if count % 2:
            _process(count - 1, 0, count)
            last0, last1 = count - 1, count - 2
        else:
            last0, last1 = count - 2, count - 1
        pltpu.make_async_copy(rows_s.at[0], accn_sh.at[dst_v.at[last0]],
                              nsem0).wait()
        pltpu.make_async_copy(w_v.at[0], accd_sh.at[dst_v.at[last0]],
                              dsem0).wait()
        pltpu.make_async_copy(rows_s.at[1], accn_sh.at[dst_v.at[last1]],
                              nsem1).wait()
        pltpu.make_async_copy(w_v.at[1], accd_sh.at[dst_v.at[last1]],
                              dsem1).wait()

    _run_phase(0, PH)
    _run_phase(PH, CHUNKS - PH)
    plsc.subcore_barrier()

    pltpu.sync_copy(accn_sh.at[sub_rows], accn_out.at[cid].at[sub_rows])
    pltpu.sync_copy(accd_sh.at[sub_rows], accd_out.at[cid].at[sub_rows])


_sc_layer = functools.partial(
    pl.kernel,
    out_type=[
        jax.ShapeDtypeStruct((NC, N_ACC, D_H), _f32),
        jax.ShapeDtypeStruct((NC, N_ACC), _f32),
    ],
    mesh=plsc.VectorSubcoreMesh(core_axis_name="c", subcore_axis_name="s"),
    compiler_params=pltpu.CompilerParams(needs_layout_passes=False,
                                         use_tc_tiling_on_sc=False),
    scratch_types=[
        pltpu.VMEM((PH, CW), jnp.int32),
        pltpu.VMEM((PH, CW), jnp.int32),
        pltpu.VMEM((N_ACC, 2), _f32),
        pltpu.VMEM((2, CW, D_H), _f32),
        pltpu.VMEM((2, CW, D_H), _f32),
        pltpu.VMEM((2, CW), _f32),
        pltpu.VMEM_SHARED((N_ACC, D_H), _f32),
        pltpu.VMEM_SHARED((N_ACC, D_H), _f32),
        pltpu.VMEM_SHARED((N_ACC,), _f32),
        pltpu.SemaphoreType.DMA,
        pltpu.SemaphoreType.DMA,
        pltpu.SemaphoreType.DMA,
        pltpu.SemaphoreType.DMA,
        pltpu.SemaphoreType.DMA,
        pltpu.SemaphoreType.DMA,
    ],
)(_sc_body)


def kernel(x, edge_index, edge_attr, W1, a1_src, a1_dst, b1, W2, a2_src,
           a2_dst, b2, Wc, bc):
    del edge_attr
    src = edge_index[0].astype(jnp.int32)
    dst = edge_index[1].astype(jnp.int32)
    loop_idx = jnp.arange(N, dtype=jnp.int32)
    pad_idx = N + (jnp.arange(E_PAD - E_TOT, dtype=jnp.int32) % N_DISC)
    src_all = jnp.concatenate([src, loop_idx, pad_idx]).reshape(NW, CHUNKS, CW)
    dst_all = jnp.concatenate([dst, loop_idx, pad_idx]).reshape(NW, CHUNKS, CW)

    zeros_n = jnp.zeros((N_ACC, D_H), _f32)
    zeros_d = jnp.zeros((N_ACC,), _f32)

    a1 = jnp.concatenate([a1_src[:, None], a1_dst[:, None]], axis=1)
    a2 = jnp.concatenate([a2_src[:, None], a2_dst[:, None]], axis=1)

    h1, st1 = _tc_dense1(x, W1, a1)
    accn1, accd1 = _sc_layer(src_all, dst_all, st1, h1, zeros_n, zeros_d)
    h2, st2 = _tc_dense2(accn1, accd1[:, :, None], b1[None, :], W2, a2)
    accn2, accd2 = _sc_layer(src_all, dst_all, st2, h2, zeros_n, zeros_d)
    out = _tc_dense3(accn2, accd2[:, :, None], b2[None, :], Wc, bc[None, :])
    return out[:N]


# speedup vs baseline: 1.3328x; 1.0780x over previous
import functools

import numpy as np

import jax
import jax.numpy as jnp
from jax import lax
from jax.experimental import pallas as pl
from jax.experimental.pallas import tpu as pltpu
from jax.experimental.pallas import tpu_sc as plsc

N = 10000
D_IN = 128
D_H = 16
N_CLS = 10
E = 320000

NC = 2
NS = 16
NW = NC * NS
CW = 128
E_TOT = E + N
CHUNKS = -(-E_TOT // (NW * CW))
E_PAD = NW * CHUNKS * CW
PH = -(-CHUNKS // 2)
N_ACC = 10112
N_DISC = N_ACC - N
ROWS_PER_SUB = N_ACC // NS
E_ROWS = E // CW
ALL_ROWS = NW * CHUNKS
TAIL_ROWS = ALL_ROWS - E_ROWS

_f32 = jnp.float32

_ZEROS_N = np.zeros((N_ACC, D_H), np.float32)
_ZEROS_D = np.zeros((N_ACC, 1), np.float32)


def _tc1_body(x_ref, w_ref, a2_ref, src2_ref, dst2_ref,
              h_ref, st_ref, sa_ref, da_ref):
    h = jnp.dot(x_ref[...], w_ref[...], preferred_element_type=_f32)
    h_ref[0:N, :] = h
    h_ref[N:N_ACC, :] = jnp.zeros((N_DISC, D_H), _f32)
    st_ref[0:N, :] = jnp.dot(h, a2_ref[...], preferred_element_type=_f32)
    st_ref[N:N_ACC, :] = jnp.zeros((N_DISC, 2), _f32)
    sa_ref[0:E_ROWS, :] = src2_ref[...]
    da_ref[0:E_ROWS, :] = dst2_ref[...]
    r = lax.broadcasted_iota(jnp.int32, (TAIL_ROWS, CW), 0)
    c = lax.broadcasted_iota(jnp.int32, (TAIL_ROWS, CW), 1)
    f = r * CW + c
    tail = jnp.where(f < N, f, N + ((f - N) % N_DISC))
    sa_ref[E_ROWS:, :] = tail
    da_ref[E_ROWS:, :] = tail


def _tc_dense1(x_pad, W1, a2, src2, dst2):
    return pl.pallas_call(
        _tc1_body,
        out_shape=[
            jax.ShapeDtypeStruct((N_ACC, D_H), _f32),
            jax.ShapeDtypeStruct((N_ACC, 2), _f32),
            jax.ShapeDtypeStruct((ALL_ROWS, CW), jnp.int32),
            jax.ShapeDtypeStruct((ALL_ROWS, CW), jnp.int32),
        ],
    )(x_pad, W1, a2, src2, dst2)


def _tc2_body(accn_ref, accd_ref, b_ref, w_ref, a2_ref, h_ref, st_ref):
    num = accn_ref[0] + accn_ref[1]
    den = accd_ref[0] + accd_ref[1]
    o = num / (den + 1e-16) + b_ref[...]
    o = jnp.maximum(o, 0.0)
    h = jnp.dot(o, w_ref[...], preferred_element_type=_f32)
    h_ref[...] = h
    st_ref[...] = jnp.dot(h, a2_ref[...], preferred_element_type=_f32)


def _tc_dense2(accn, accd, b1, W2, a2):
    return pl.pallas_call(
        _tc2_body,
        out_shape=[
            jax.ShapeDtypeStruct((N_ACC, D_H), _f32),
            jax.ShapeDtypeStruct((N_ACC, 2), _f32),
        ],
    )(accn, accd, b1, W2, a2)


def _tc3_body(accn_ref, accd_ref, b_ref, wc_ref, bc_ref, out_ref):
    num = accn_ref[0] + accn_ref[1]
    den = accd_ref[0] + accd_ref[1]
    o = num / (den + 1e-16) + b_ref[...]
    logits = jnp.dot(o, wc_ref[...], preferred_element_type=_f32) + bc_ref[...]
    m = jnp.max(logits, axis=1, keepdims=True)
    z = logits - m
    lse = jnp.log(jnp.sum(jnp.exp(z), axis=1, keepdims=True))
    out_ref[...] = z - lse


def _tc_dense3(accn, accd, b2, Wc, bc):
    return pl.pallas_call(
        _tc3_body,
        out_shape=jax.ShapeDtypeStruct((N_ACC, N_CLS), _f32),
    )(accn, accd, b2, Wc, bc)


def _sc_body(src_hbm, dst_hbm, st_hbm, h_hbm, zn_hbm, zd_hbm,
             accn_out, accd_out,
             src_v, dst_v, st_v, rows_v, rows_s, w_v,
             h_sh, accn_sh, accd_sh,
             gsem0, gsem1, nsem0, nsem1, dsem0, dsem1):
    cid = lax.axis_index("c")
    sid = lax.axis_index("s")
    wid = sid * NC + cid
    sub_rows = pl.ds(sid * ROWS_PER_SUB, ROWS_PER_SUB)

    pltpu.sync_copy(st_hbm, st_v)
    pltpu.sync_copy(h_hbm.at[sub_rows], h_sh.at[sub_rows])
    pltpu.sync_copy(zn_hbm.at[sub_rows], accn_sh.at[sub_rows])
    pltpu.sync_copy(zd_hbm.at[sub_rows], accd_sh.at[sub_rows])
    plsc.subcore_barrier()

    zeros16 = jnp.zeros((16,), jnp.int32)
    ones16 = jnp.ones((16,), jnp.int32)
    gsems = (gsem0, gsem1)
    nsems = (nsem0, nsem1)
    dsems = (dsem0, dsem1)

    def _process(jj, b, count):
        pltpu.make_async_copy(h_sh.at[src_v.at[jj]], rows_v.at[b],
                              gsems[b]).wait()
        nb = 1 - b

        @pl.when(jj < count - 1)
        def _():
            pltpu.async_copy(h_sh.at[src_v.at[jj + 1]], rows_v.at[nb],
                             gsems[nb])

        @pl.when(jj >= 2)
        def _():
            pltpu.make_async_copy(rows_s.at[b], accn_sh.at[dst_v.at[jj - 2]],
                                  nsems[b]).wait()
            pltpu.make_async_copy(w_v.at[b], accd_sh.at[dst_v.at[jj - 2]],
                                  dsems[b]).wait()

        for k in range(CW // 16):
            srcs = src_v[jj, pl.ds(k * 16, 16)]
            dsts = dst_v[jj, pl.ds(k * 16, 16)]
            sv = plsc.load_gather(st_v, [srcs, zeros16])
            tv = plsc.load_gather(st_v, [dsts, ones16])
            z = sv + tv
            z = jnp.where(z >= 0.0, z, 0.2 * z)
            wv = jnp.exp(z)
            plsc.store_scatter(
                w_v.at[b],
                [jnp.arange(k * 16, k * 16 + 16, dtype=jnp.int32), zeros16],
                wv)
            for i in range(16):
                wsc = wv[i]
                r = k * 16 + i
                rows_s[b, r, :] = rows_v[b, r, :] * wsc
        pltpu.async_copy(rows_s.at[b], accn_sh.at[dst_v.at[jj]],
                         nsems[b], add=True)
        pltpu.async_copy(w_v.at[b], accd_sh.at[dst_v.at[jj]],
                         dsems[b], add=True)

    def _run_phase(start, count):
        pltpu.sync_copy(src_hbm.at[wid].at[pl.ds(start, count)],
                        src_v.at[pl.ds(0, count)])
        pltpu.sync_copy(dst_hbm.at[wid].at[pl.ds(start, count)],
                        dst_v.at[pl.ds(0, count)])
        pltpu.async_copy(h_sh.at[src_v.at[0]], rows_v.at[0], gsem0)

        def cbody(p, carry):
            _process(2 * p, 0, count)
            _process(2 * p + 1, 1, count)
            return carry

        lax.fori_loop(0, count // 2, cbody, 0)
        if count % 2:
            _process(count - 1, 0, count)
            last0, last1 = count - 1, count - 2
        else:
            last0, last1 = count - 2, count - 1
        pltpu.make_async_copy(rows_s.at[0], accn_sh.at[dst_v.at[last0]],
                              nsem0).wait()
        pltpu.make_async_copy(w_v.at[0], accd_sh.at[dst_v.at[last0]],
                              dsem0).wait()
        pltpu.make_async_copy(rows_s.at[1], accn_sh.at[dst_v.at[last1]],
                              nsem1).wait()
        pltpu.make_async_copy(w_v.at[1], accd_sh.at[dst_v.at[last1]],
                              dsem1).wait()

    _run_phase(0, PH)
    _run_phase(PH, CHUNKS - PH)
    plsc.subcore_barrier()

    pltpu.sync_copy(accn_sh.at[sub_rows], accn_out.at[cid].at[sub_rows])
    pltpu.sync_copy(accd_sh.at[sub_rows], accd_out.at[cid].at[sub_rows])


_sc_layer = functools.partial(
    pl.kernel,
    out_type=[
        jax.ShapeDtypeStruct((NC, N_ACC, D_H), _f32),
        jax.ShapeDtypeStruct((NC, N_ACC, 1), _f32),
    ],
    mesh=plsc.VectorSubcoreMesh(core_axis_name="c", subcore_axis_name="s"),
    compiler_params=pltpu.CompilerParams(needs_layout_passes=False,
                                         use_tc_tiling_on_sc=False),
    scratch_types=[
        pltpu.VMEM((PH, CW), jnp.int32),
        pltpu.VMEM((PH, CW), jnp.int32),
        pltpu.VMEM((N_ACC, 2), _f32),
        pltpu.VMEM((2, CW, D_H), _f32),
        pltpu.VMEM((2, CW, D_H), _f32),
        pltpu.VMEM((2, CW, 1), _f32),
        pltpu.VMEM_SHARED((N_ACC, D_H), _f32),
        pltpu.VMEM_SHARED((N_ACC, D_H), _f32),
        pltpu.VMEM_SHARED((N_ACC, 1), _f32),
        pltpu.SemaphoreType.DMA,
        pltpu.SemaphoreType.DMA,
        pltpu.SemaphoreType.DMA,
        pltpu.SemaphoreType.DMA,
        pltpu.SemaphoreType.DMA,
        pltpu.SemaphoreType.DMA,
    ],
)(_sc_body)


def kernel(x, edge_index, edge_attr, W1, a1_src, a1_dst, b1, W2, a2_src,
           a2_dst, b2, Wc, bc):
    del edge_attr
    src2 = edge_index[0].astype(jnp.int32).reshape(E_ROWS, CW)
    dst2 = edge_index[1].astype(jnp.int32).reshape(E_ROWS, CW)

    zeros_n = jnp.asarray(_ZEROS_N)
    zeros_d = jnp.asarray(_ZEROS_D)

    a1 = jnp.concatenate([a1_src[:, None], a1_dst[:, None]], axis=1)
    a2 = jnp.concatenate([a2_src[:, None], a2_dst[:, None]], axis=1)

    h1, st1, src_a, dst_a = _tc_dense1(x, W1, a1, src2, dst2)
    src_all = src_a.reshape(NW, CHUNKS, CW)
    dst_all = dst_a.reshape(NW, CHUNKS, CW)
    accn1, accd1 = _sc_layer(src_all, dst_all, st1, h1, zeros_n, zeros_d)
    h2, st2 = _tc_dense2(accn1, accd1, b1[None, :], W2, a2)
    accn2, accd2 = _sc_layer(src_all, dst_all, st2, h2, zeros_n, zeros_d)
    out = _tc_dense3(accn2, accd2, b2[None, :], Wc, bc[None, :])
    return out[:N]


# speedup vs baseline: 1.3349x; 1.0016x over previous
import functools

import numpy as np

import jax
import jax.numpy as jnp
from jax import lax
from jax.experimental import pallas as pl
from jax.experimental.pallas import tpu as pltpu
from jax.experimental.pallas import tpu_sc as plsc

N = 10000
D_IN = 128
D_H = 16
N_CLS = 10
E = 320000

NC = 2
NS = 16
NW = NC * NS
CW = 128
E_TOT = E + N
CHUNKS = -(-E_TOT // (NW * CW))
E_PAD = NW * CHUNKS * CW
PH = -(-CHUNKS // 2)
N_ACC = 10112
N_DISC = N_ACC - N
ROWS_PER_SUB = N_ACC // NS
E_ROWS = E // CW
ALL_ROWS = NW * CHUNKS
TAIL_ROWS = ALL_ROWS - E_ROWS

_f32 = jnp.float32

_ZEROS_N = np.zeros((N_ACC, D_H), np.float32)
_ZEROS_D = np.zeros((N_ACC, 1), np.float32)


def _tc1_body(x_ref, w_ref, a2_ref, src2_ref, dst2_ref,
              h_ref, st_ref, sa_ref, da_ref):
    h = jnp.dot(x_ref[...], w_ref[...], preferred_element_type=_f32)
    h_ref[0:N, :] = h
    h_ref[N:N_ACC, :] = jnp.zeros((N_DISC, D_H), _f32)
    st_ref[0:N, :] = jnp.dot(h, a2_ref[...], preferred_element_type=_f32)
    st_ref[N:N_ACC, :] = jnp.zeros((N_DISC, 2), _f32)
    sa_ref[0:E_ROWS, :] = src2_ref[...]
    da_ref[0:E_ROWS, :] = dst2_ref[...]
    r = lax.broadcasted_iota(jnp.int32, (TAIL_ROWS, CW), 0)
    c = lax.broadcasted_iota(jnp.int32, (TAIL_ROWS, CW), 1)
    f = r * CW + c
    tail = jnp.where(f < N, f, N + ((f - N) % N_DISC))
    sa_ref[E_ROWS:, :] = tail
    da_ref[E_ROWS:, :] = tail


def _tc_dense1(x_pad, W1, a2, src2, dst2):
    return pl.pallas_call(
        _tc1_body,
        out_shape=[
            jax.ShapeDtypeStruct((N_ACC, D_H), _f32),
            jax.ShapeDtypeStruct((N_ACC, 2), _f32),
            jax.ShapeDtypeStruct((ALL_ROWS, CW), jnp.int32),
            jax.ShapeDtypeStruct((ALL_ROWS, CW), jnp.int32),
        ],
    )(x_pad, W1, a2, src2, dst2)


def _tc2_body(accn_ref, accd_ref, b_ref, w_ref, a2_ref, h_ref, st_ref):
    num = accn_ref[0] + accn_ref[1]
    den = accd_ref[0] + accd_ref[1]
    o = num / (den + 1e-16) + b_ref[...]
    o = jnp.maximum(o, 0.0)
    h = jnp.dot(o, w_ref[...], preferred_element_type=_f32)
    h_ref[...] = h
    st_ref[...] = jnp.dot(h, a2_ref[...], preferred_element_type=_f32)


def _tc_dense2(accn, accd, b1, W2, a2):
    return pl.pallas_call(
        _tc2_body,
        out_shape=[
            jax.ShapeDtypeStruct((N_ACC, D_H), _f32),
            jax.ShapeDtypeStruct((N_ACC, 2), _f32),
        ],
    )(accn, accd, b1, W2, a2)


def _tc3_body(accn_ref, accd_ref, b_ref, wc_ref, bc_ref, out_ref):
    num = accn_ref[0] + accn_ref[1]
    den = accd_ref[0] + accd_ref[1]
    o = num / (den + 1e-16) + b_ref[...]
    logits = jnp.dot(o, wc_ref[...], preferred_element_type=_f32) + bc_ref[...]
    m = jnp.max(logits, axis=1, keepdims=True)
    z = logits - m
    lse = jnp.log(jnp.sum(jnp.exp(z), axis=1, keepdims=True))
    out_ref[...] = z - lse


def _tc_dense3(accn, accd, b2, Wc, bc):
    return pl.pallas_call(
        _tc3_body,
        out_shape=jax.ShapeDtypeStruct((N_ACC, N_CLS), _f32),
    )(accn, accd, b2, Wc, bc)


def _sc_body(src_hbm, dst_hbm, st_hbm, h_hbm, zn_hbm, zd_hbm,
             accn_out, accd_out,
             src_v, dst_v, st_v, rows_v, rows_s, w_v,
             h_sh, accn_sh, accd_sh,
             gsem0, gsem1, nsem0, nsem1, dsem0, dsem1):
    cid = lax.axis_index("c")
    sid = lax.axis_index("s")
    wid = sid * NC + cid
    sub_rows = pl.ds(sid * ROWS_PER_SUB, ROWS_PER_SUB)

    pltpu.sync_copy(st_hbm, st_v)
    pltpu.sync_copy(h_hbm.at[sub_rows], h_sh.at[sub_rows])
    pltpu.sync_copy(zn_hbm.at[sub_rows], accn_sh.at[sub_rows])
    pltpu.sync_copy(zd_hbm.at[sub_rows], accd_sh.at[sub_rows])
    plsc.subcore_barrier()

    zeros16 = jnp.zeros((16,), jnp.int32)
    ones16 = jnp.ones((16,), jnp.int32)
    gsems = (gsem0, gsem1)
    nsems = (nsem0, nsem1)
    dsems = (dsem0, dsem1)

    def _process(jj, b, count):
        pltpu.make_async_copy(h_sh.at[src_v.at[jj]], rows_v.at[b],
                              gsems[b]).wait()
        nb = 1 - b

        @pl.when(jj < count - 1)
        def _():
            pltpu.async_copy(h_sh.at[src_v.at[jj + 1]], rows_v.at[nb],
                             gsems[nb])

        @pl.when(jj >= 2)
        def _():
            pltpu.make_async_copy(rows_s.at[b], accn_sh.at[dst_v.at[jj - 2]],
                                  nsems[b]).wait()
            pltpu.make_async_copy(w_v.at[b], accd_sh.at[dst_v.at[jj - 2]],
                                  dsems[b]).wait()

        for k in range(CW // 16):
            srcs = src_v[jj, pl.ds(k * 16, 16)]
            dsts = dst_v[jj, pl.ds(k * 16, 16)]
            sv = plsc.load_gather(st_v, [srcs, zeros16])
            tv = plsc.load_gather(st_v, [dsts, ones16])
            z = sv + tv
            z = jnp.where(z >= 0.0, z, 0.2 * z)
            wv = jnp.exp(z)
            plsc.store_scatter(
                w_v.at[b],
                [jnp.arange(k * 16, k * 16 + 16, dtype=jnp.int32), zeros16],
                wv)
            for i in range(16):
                wsc = wv[i]
                r = k * 16 + i
                rows_s[b, r, :] = rows_v[b, r, :] * wsc
        pltpu.async_copy(rows_s.at[b], accn_sh.at[dst_v.at[jj]],
                         nsems[b], add=True)
        pltpu.async_copy(w_v.at[b], accd_sh.at[dst_v.at[jj]],
                         dsems[b], add=True)

    def _run_phase(start, count):
        pltpu.sync_copy(src_hbm.at[pl.ds(wid * CHUNKS + start, count)],
                        src_v.at[pl.ds(0, count)])
        pltpu.sync_copy(dst_hbm.at[pl.ds(wid * CHUNKS + start, count)],
                        dst_v.at[pl.ds(0, count)])
        pltpu.async_copy(h_sh.at[src_v.at[0]], rows_v.at[0], gsem0)

        def cbody(p, carry):
            _process(2 * p, 0, count)
            _process(2 * p + 1, 1, count)
            return carry

        lax.fori_loop(0, count // 2, cbody, 0)
        if count % 2:
            _process(count - 1, 0, count)
            last0, last1 = count - 1, count - 2
        else:
            last0, last1 = count - 2, count - 1
        pltpu.make_async_copy(rows_s.at[0], accn_sh.at[dst_v.at[last0]],
                              nsem0).wait()
        pltpu.make_async_copy(w_v.at[0], accd_sh.at[dst_v.at[last0]],
                              dsem0).wait()
        pltpu.make_async_copy(rows_s.at[1], accn_sh.at[dst_v.at[last1]],
                              nsem1).wait()
        pltpu.make_async_copy(w_v.at[1], accd_sh.at[dst_v.at[last1]],
                              dsem1).wait()

    _run_phase(0, PH)
    _run_phase(PH, CHUNKS - PH)
    plsc.subcore_barrier()

    pltpu.sync_copy(accn_sh.at[sub_rows], accn_out.at[cid].at[sub_rows])
    pltpu.sync_copy(accd_sh.at[sub_rows], accd_out.at[cid].at[sub_rows])


_sc_layer = functools.partial(
    pl.kernel,
    out_type=[
        jax.ShapeDtypeStruct((NC, N_ACC, D_H), _f32),
        jax.ShapeDtypeStruct((NC, N_ACC, 1), _f32),
    ],
    mesh=plsc.VectorSubcoreMesh(core_axis_name="c", subcore_axis_name="s"),
    compiler_params=pltpu.CompilerParams(needs_layout_passes=False,
                                         use_tc_tiling_on_sc=False),
    scratch_types=[
        pltpu.VMEM((PH, CW), jnp.int32),
        pltpu.VMEM((PH, CW), jnp.int32),
        pltpu.VMEM((N_ACC, 2), _f32),
        pltpu.VMEM((2, CW, D_H), _f32),
        pltpu.VMEM((2, CW, D_H), _f32),
        pltpu.VMEM((2, CW, 1), _f32),
        pltpu.VMEM_SHARED((N_ACC, D_H), _f32),
        pltpu.VMEM_SHARED((N_ACC, D_H), _f32),
        pltpu.VMEM_SHARED((N_ACC, 1), _f32),
        pltpu.SemaphoreType.DMA,
        pltpu.SemaphoreType.DMA,
        pltpu.SemaphoreType.DMA,
        pltpu.SemaphoreType.DMA,
        pltpu.SemaphoreType.DMA,
        pltpu.SemaphoreType.DMA,
    ],
)(_sc_body)


def kernel(x, edge_index, edge_attr, W1, a1_src, a1_dst, b1, W2, a2_src,
           a2_dst, b2, Wc, bc):
    del edge_attr
    src2 = edge_index[0].astype(jnp.int32).reshape(E_ROWS, CW)
    dst2 = edge_index[1].astype(jnp.int32).reshape(E_ROWS, CW)

    zeros_n = jnp.asarray(_ZEROS_N)
    zeros_d = jnp.asarray(_ZEROS_D)

    a1 = jnp.concatenate([a1_src[:, None], a1_dst[:, None]], axis=1)
    a2 = jnp.concatenate([a2_src[:, None], a2_dst[:, None]], axis=1)

    h1, st1, src_all, dst_all = _tc_dense1(x, W1, a1, src2, dst2)
    accn1, accd1 = _sc_layer(src_all, dst_all, st1, h1, zeros_n, zeros_d)
    h2, st2 = _tc_dense2(accn1, accd1, b1[None, :], W2, a2)
    accn2, accd2 = _sc_layer(src_all, dst_all, st2, h2, zeros_n, zeros_d)
    out = _tc_dense3(accn2, accd2, b2[None, :], Wc, bc[None, :])
    return out[:N]
